# baseline (device time: 199585 ns/iter reference)
import jax
import jax.numpy as jnp
from jax import lax
from jax.experimental import pallas as pl
from jax.experimental.pallas import tpu as pltpu

CHUNKS = 8


def kernel(Q, K, V):
    b, s, h, d = K.shape
    bh = b * h
    half = s // 2
    cs = half // CHUNKS
    gp = bh // CHUNKS
    scale = d ** -0.5

    Qt = Q.transpose(0, 2, 1, 3).reshape(bh, s, d)
    Kt = K.transpose(0, 2, 1, 3).reshape(bh, s, d)
    Vt = V.transpose(0, 2, 1, 3).reshape(bh, s, d)

    def body(q_ref, k_ref, v_ref, o_ref, kg, vg, sx, rx, sy, ry):
        my_x = lax.axis_index("x")
        my_y = lax.axis_index("y")
        nx = (1 - my_x, my_y)
        ny = (my_x, 1 - my_y)

        barrier = pltpu.get_barrier_semaphore()
        for nbr in (nx, ny):
            pl.semaphore_signal(barrier, inc=1, device_id=nbr,
                                device_id_type=pl.DeviceIdType.MESH)
        pl.semaphore_wait(barrier, 2)

        my_off = my_y * half
        tensors = ((k_ref, kg), (v_ref, vg))

        x_sends = []
        for c in range(CHUNKS):
            for t, (src, dst) in enumerate(tensors):
                r = pltpu.make_async_remote_copy(
                    src_ref=src.at[:, pl.ds(my_off + c * cs, cs)],
                    dst_ref=dst.at[:, pl.ds(c * cs, cs)],
                    send_sem=sx.at[t, c],
                    recv_sem=rx.at[t, c],
                    device_id=nx,
                    device_id_type=pl.DeviceIdType.MESH,
                )
                r.start()
                x_sends.append(r)

        def stage_a(i, carry):
            q = q_ref[i] * scale
            S = lax.dot_general(q, k_ref[i], (((1,), (1,)), ((), ())),
                                preferred_element_type=jnp.float32)
            m = jnp.max(S, axis=1, keepdims=True)
            p = jnp.exp(S - m)
            l = jnp.sum(p, axis=1, keepdims=True)
            acc = lax.dot_general(p, v_ref[i], (((1,), (0,)), ((), ())),
                                  preferred_element_type=jnp.float32)
            o_ref[i, :, 0:d] = acc
            o_ref[i, :, d:d + 1] = m
            o_ref[i, :, d + 1:d + 2] = l
            return carry

        def make_merge(lo, final):
            def merge(i, carry):
                q = q_ref[i] * scale
                S = lax.dot_general(q, kg[i, lo:lo + half],
                                    (((1,), (1,)), ((), ())),
                                    preferred_element_type=jnp.float32)
                mb = jnp.max(S, axis=1, keepdims=True)
                m_old = o_ref[i, :, d:d + 1]
                l_old = o_ref[i, :, d + 1:d + 2]
                acc_old = o_ref[i, :, 0:d]
                m_new = jnp.maximum(m_old, mb)
                p = jnp.exp(S - m_new)
                alpha = jnp.exp(m_old - m_new)
                l_new = l_old * alpha + jnp.sum(p, axis=1, keepdims=True)
                pv = lax.dot_general(p, vg[i, lo:lo + half],
                                     (((1,), (0,)), ((), ())),
                                     preferred_element_type=jnp.float32)
                acc = acc_old * alpha + pv
                if final:
                    o_ref[i, :, 0:d] = acc / l_new
                else:
                    o_ref[i, :, 0:d] = acc
                    o_ref[i, :, d:d + 1] = m_new
                    o_ref[i, :, d + 1:d + 2] = l_new
                return carry

            return merge

        merge_x = make_merge(0, final=False)
        merge_y = make_merge(half, final=True)

        y_fwds = []
        for c in range(CHUNKS):
            for t, (src, dst) in enumerate(tensors):
                recv = pltpu.make_async_remote_copy(
                    src_ref=src.at[:, pl.ds(my_off + c * cs, cs)],
                    dst_ref=dst.at[:, pl.ds(c * cs, cs)],
                    send_sem=sx.at[t, c],
                    recv_sem=rx.at[t, c],
                    device_id=nx,
                    device_id_type=pl.DeviceIdType.MESH,
                )
                recv.wait_recv()
                f = pltpu.make_async_remote_copy(
                    src_ref=dst.at[:, pl.ds(c * cs, cs)],
                    dst_ref=dst.at[:, pl.ds(half + c * cs, cs)],
                    send_sem=sy.at[t, c],
                    recv_sem=ry.at[t, c],
                    device_id=ny,
                    device_id_type=pl.DeviceIdType.MESH,
                )
                f.start()
                y_fwds.append(f)
            lax.fori_loop(c * gp, (c + 1) * gp, stage_a, 0)

        for c in range(CHUNKS):
            for t, (src, dst) in enumerate(tensors):
                rv = pltpu.make_async_remote_copy(
                    src_ref=src.at[:, pl.ds(my_off + c * cs, cs)],
                    dst_ref=dst.at[:, pl.ds(half + c * cs, cs)],
                    send_sem=sy.at[t, c],
                    recv_sem=ry.at[t, c],
                    device_id=ny,
                    device_id_type=pl.DeviceIdType.MESH,
                )
                rv.wait_recv()
            lax.fori_loop(c * gp, (c + 1) * gp, merge_x, 0)

        lax.fori_loop(0, bh, merge_y, 0)

        for r in x_sends:
            r.wait_send()
        for f in y_fwds:
            f.wait_send()

    out = pl.pallas_call(
        body,
        out_shape=jax.ShapeDtypeStruct((bh, s, 2 * d), jnp.float32),
        in_specs=[pl.BlockSpec(memory_space=pltpu.VMEM)] * 3,
        out_specs=pl.BlockSpec(memory_space=pltpu.VMEM),
        scratch_shapes=[
            pltpu.VMEM((bh, s, d), jnp.float32),
            pltpu.VMEM((bh, s, d), jnp.float32),
            pltpu.SemaphoreType.DMA((2, CHUNKS)),
            pltpu.SemaphoreType.DMA((2, CHUNKS)),
            pltpu.SemaphoreType.DMA((2, CHUNKS)),
            pltpu.SemaphoreType.DMA((2, CHUNKS)),
        ],
        compiler_params=pltpu.CompilerParams(
            collective_id=0, vmem_limit_bytes=64 * 1024 * 1024),
    )(Qt, Kt, Vt)

    return out[:, :, :d].reshape(b, h, s, d).transpose(0, 2, 1, 3)


# device time: 151361 ns/iter; 1.3186x vs baseline; 1.3186x over previous
import jax
import jax.numpy as jnp
from jax import lax
from jax.experimental import pallas as pl
from jax.experimental.pallas import tpu as pltpu

CHUNKS = 8


def kernel(Q, K, V):
    b, s, h, d = K.shape
    bh = b * h
    hhalf = bh // 2
    ch = hhalf // CHUNKS
    scale = d ** -0.5

    Qt = Q.transpose(0, 2, 1, 3).reshape(bh, s, d)
    Kt = K.transpose(0, 2, 1, 3).reshape(bh, s, d)
    Vt = V.transpose(0, 2, 1, 3).reshape(bh, s, d)

    def body(q_ref, k_ref, v_ref, o_ref, kg, vg, sx, rx, sy, ry):
        my_x = lax.axis_index("x")
        my_y = lax.axis_index("y")
        nx = (1 - my_x, my_y)
        ny = (my_x, 1 - my_y)

        barrier = pltpu.get_barrier_semaphore()
        for nbr in (nx, ny):
            pl.semaphore_signal(barrier, inc=1, device_id=nbr,
                                device_id_type=pl.DeviceIdType.MESH)
        pl.semaphore_wait(barrier, 2)

        base_x = my_y * hhalf
        base_y = (1 - my_y) * hhalf
        tensors = ((k_ref, kg), (v_ref, vg))

        x_sends = []
        for c in range(CHUNKS):
            for t, (src, dst) in enumerate(tensors):
                r = pltpu.make_async_remote_copy(
                    src_ref=src.at[pl.ds(base_x + c * ch, ch)],
                    dst_ref=dst.at[pl.ds(base_x + c * ch, ch)],
                    send_sem=sx.at[t, c],
                    recv_sem=rx.at[t, c],
                    device_id=nx,
                    device_id_type=pl.DeviceIdType.MESH,
                )
                r.start()
                x_sends.append(r)

        def one(i, carry):
            q = q_ref[i] * scale
            S1 = lax.dot_general(q, k_ref[i], (((1,), (1,)), ((), ())),
                                 preferred_element_type=jnp.float32)
            S2 = lax.dot_general(q, kg[i], (((1,), (1,)), ((), ())),
                                 preferred_element_type=jnp.float32)
            m = jnp.maximum(jnp.max(S1, axis=1, keepdims=True),
                            jnp.max(S2, axis=1, keepdims=True))
            p1 = jnp.exp(S1 - m)
            p2 = jnp.exp(S2 - m)
            l = (jnp.sum(p1, axis=1, keepdims=True)
                 + jnp.sum(p2, axis=1, keepdims=True))
            acc = (lax.dot_general(p1, v_ref[i], (((1,), (0,)), ((), ())),
                                   preferred_element_type=jnp.float32)
                   + lax.dot_general(p2, vg[i], (((1,), (0,)), ((), ())),
                                     preferred_element_type=jnp.float32))
            o_ref[i] = acc / l
            return carry

        y_fwds = []
        for c in range(CHUNKS):
            for t, (src, dst) in enumerate(tensors):
                recv = pltpu.make_async_remote_copy(
                    src_ref=src.at[pl.ds(base_x + c * ch, ch)],
                    dst_ref=dst.at[pl.ds(base_x + c * ch, ch)],
                    send_sem=sx.at[t, c],
                    recv_sem=rx.at[t, c],
                    device_id=nx,
                    device_id_type=pl.DeviceIdType.MESH,
                )
                recv.wait_recv()
                f = pltpu.make_async_remote_copy(
                    src_ref=dst.at[pl.ds(base_x + c * ch, ch)],
                    dst_ref=dst.at[pl.ds(base_x + c * ch, ch)],
                    send_sem=sy.at[t, c],
                    recv_sem=ry.at[t, c],
                    device_id=ny,
                    device_id_type=pl.DeviceIdType.MESH,
                )
                f.start()
                y_fwds.append(f)
            lax.fori_loop(base_x + c * ch, base_x + (c + 1) * ch, one, 0)

        for c in range(CHUNKS):
            for t, (src, dst) in enumerate(tensors):
                rv = pltpu.make_async_remote_copy(
                    src_ref=src.at[pl.ds(base_y + c * ch, ch)],
                    dst_ref=dst.at[pl.ds(base_y + c * ch, ch)],
                    send_sem=sy.at[t, c],
                    recv_sem=ry.at[t, c],
                    device_id=ny,
                    device_id_type=pl.DeviceIdType.MESH,
                )
                rv.wait_recv()
            lax.fori_loop(base_y + c * ch, base_y + (c + 1) * ch, one, 0)

        for r in x_sends:
            r.wait_send()
        for f in y_fwds:
            f.wait_send()

    out = pl.pallas_call(
        body,
        out_shape=jax.ShapeDtypeStruct((bh, s, d), jnp.float32),
        in_specs=[pl.BlockSpec(memory_space=pltpu.VMEM)] * 3,
        out_specs=pl.BlockSpec(memory_space=pltpu.VMEM),
        scratch_shapes=[
            pltpu.VMEM((bh, s, d), jnp.float32),
            pltpu.VMEM((bh, s, d), jnp.float32),
            pltpu.SemaphoreType.DMA((2, CHUNKS)),
            pltpu.SemaphoreType.DMA((2, CHUNKS)),
            pltpu.SemaphoreType.DMA((2, CHUNKS)),
            pltpu.SemaphoreType.DMA((2, CHUNKS)),
        ],
        compiler_params=pltpu.CompilerParams(
            collective_id=0, vmem_limit_bytes=64 * 1024 * 1024),
    )(Qt, Kt, Vt)

    return out.reshape(b, h, s, d).transpose(0, 2, 1, 3)
